# baseline (speedup 1.0000x reference)
import functools

import jax
import jax.numpy as jnp
from jax.experimental import pallas as pl
from jax.experimental.pallas import tpu as pltpu

_K_H = 4


def _router_kernel(x_ref, gw_ref, b_ref, wt_ref):
    x = x_ref[...]
    gw = gw_ref[...]
    logits = jax.lax.dot_general(
        x, gw, (((1,), (1,)), ((), ())), preferred_element_type=jnp.float32)
    logits = logits + b_ref[...]
    t, e = logits.shape
    e_iota = jax.lax.broadcasted_iota(jnp.int32, (t, e), 1)
    m1 = jnp.max(logits, axis=1, keepdims=True)
    i1 = jnp.min(jnp.where(logits == m1, e_iota, e), axis=1, keepdims=True)
    masked = jnp.where(e_iota == i1, -jnp.inf, logits)
    m2 = jnp.max(masked, axis=1, keepdims=True)
    i2 = jnp.min(jnp.where(masked == m2, e_iota, e), axis=1, keepdims=True)
    w1 = 1.0 / (1.0 + jnp.exp(m2 - m1))
    w2 = 1.0 - w1
    wt_ref[...] = (jnp.where(e_iota == i1, w1, 0.0)
                   + jnp.where(e_iota == i2, w2, 0.0))


def _ffn_kernel(es_ref, na_ref, x_ref, wt_ref, g_ref, u_ref, d_ref, o_ref,
                gacc_ref, uacc_ref):
    del es_ref, na_ref
    s = pl.program_id(0)
    k = pl.program_id(1)
    x = x_ref[...]
    gp = jnp.dot(x, g_ref[0], preferred_element_type=jnp.float32)
    up = jnp.dot(x, u_ref[0], preferred_element_type=jnp.float32)

    @pl.when(k == 0)
    def _():
        gacc_ref[...] = gp
        uacc_ref[...] = up

    @pl.when(k != 0)
    def _():
        gacc_ref[...] += gp
        uacc_ref[...] += up

    @pl.when(k == _K_H - 1)
    def _():
        g = gacc_ref[...]
        act = g * jax.nn.sigmoid(g) * uacc_ref[...]
        part = jnp.dot(act, d_ref[0], preferred_element_type=jnp.float32)
        wt = wt_ref[...]
        col = jax.lax.broadcasted_iota(jnp.int32, wt.shape, 1)
        w = jnp.sum(jnp.where(col == s, wt, 0.0), axis=1, keepdims=True)
        contrib = part * w

        @pl.when(s == 0)
        def _():
            o_ref[...] = contrib

        @pl.when(s != 0)
        def _():
            o_ref[...] += contrib


@functools.partial(jax.jit, static_argnames=())
def kernel(x, gate_w, expert_bias, gate_proj, up_proj, down_proj):
    b, s_len, h = x.shape
    e = gate_proj.shape[0]
    inner = gate_proj.shape[2]
    t = b * s_len
    x2 = x.reshape(t, h)

    wt = pl.pallas_call(
        _router_kernel,
        out_shape=jax.ShapeDtypeStruct((t, e), jnp.float32),
    )(x2, gate_w, expert_bias.reshape(1, e))

    active = jnp.any(wt > 0.0, axis=0)
    num_active = jnp.sum(active.astype(jnp.int32))
    order = jnp.argsort(jnp.logical_not(active), stable=True).astype(jnp.int32)
    last = order[num_active - 1]
    steps = jnp.arange(e, dtype=jnp.int32)
    es = jnp.where(steps < num_active, order, last)
    wt_sched = jnp.where(steps[None, :] < num_active, wt[:, es], 0.0)
    na = num_active.reshape(1)

    k_h = _K_H
    bh = h // k_h

    def gu_idx(s, k, es, na):
        fk = jnp.where(s < na[0], k, k_h - 1)
        return (es[s], fk, 0)

    def d_idx(s, k, es, na):
        return (es[s], 0, 0)

    def x_idx(s, k, es, na):
        return (0, jnp.where(s < na[0], k, k_h - 1))

    out = pl.pallas_call(
        _ffn_kernel,
        grid_spec=pltpu.PrefetchScalarGridSpec(
            num_scalar_prefetch=2,
            grid=(e, k_h),
            in_specs=[
                pl.BlockSpec((t, bh), x_idx),
                pl.BlockSpec((t, e), lambda s, k, es, na: (0, 0)),
                pl.BlockSpec((1, bh, inner), gu_idx),
                pl.BlockSpec((1, bh, inner), gu_idx),
                pl.BlockSpec((1, inner, h), d_idx),
            ],
            out_specs=pl.BlockSpec((t, h), lambda s, k, es, na: (0, 0)),
            scratch_shapes=[
                pltpu.VMEM((t, inner), jnp.float32),
                pltpu.VMEM((t, inner), jnp.float32),
            ],
        ),
        out_shape=jax.ShapeDtypeStruct((t, h), jnp.float32),
    )(es, na, x2, wt_sched, gate_proj, up_proj, down_proj)

    return out.reshape(b, s_len, h)
